# baseline (speedup 1.0000x reference)
import numpy as np

import jax
import jax.numpy as jnp
from jax.experimental import pallas as pl
from jax.experimental.pallas import tpu as pltpu

_B_BLK = 256
_V_BLK = 2048

_KS0 = np.uint32(0)
_KS1 = np.uint32(42)
_KS2 = np.uint32(np.uint32(0x1BD11BDA) ^ np.uint32(42))
_ROT = ((13, 15, 26, 6), (17, 29, 16, 24))
_TINY = np.float32(np.finfo(np.float32).tiny)


def _gumbel_bits(lin_u32):
    ks = (_KS0, _KS1, _KS2)
    x0 = jnp.zeros_like(lin_u32)
    x1 = lin_u32 + _KS1

    for r in range(5):
        for d in _ROT[r % 2]:
            x0 = x0 + x1
            x1 = (x1 << np.uint32(d)) | (x1 >> np.uint32(32 - d))
            x1 = x0 ^ x1
        x0 = x0 + ks[(r + 1) % 3]
        x1 = x1 + ks[(r + 2) % 3] + np.uint32(r + 1)

    bits = x0 ^ x1
    fb = (bits >> np.uint32(9)) | np.uint32(0x3F800000)
    u = jax.lax.bitcast_convert_type(fb, jnp.float32) - jnp.float32(1.0)
    u = jnp.maximum(_TINY, u * (np.float32(1.0) - _TINY) + _TINY)
    return -jnp.log(-jnp.log(u))


def _body(n_bb, v_total, logits_t_ref, out_t_ref, best_val, best_idx):
    s = pl.program_id(0)
    vb = pl.program_id(1)
    par = jax.lax.rem(s, 2)

    vgl = (
        jax.lax.broadcasted_iota(jnp.int32, (_V_BLK, _B_BLK), 0) + vb * _V_BLK
    )

    @pl.when(s < n_bb)
    def _sample():
        cur_v = best_val.at[par]
        cur_i = best_idx.at[par]

        @pl.when(vb == 0)
        def _init():
            cur_v[...] = jnp.full((1, _B_BLK), -jnp.inf, jnp.float32)
            cur_i[...] = jnp.zeros((1, _B_BLK), jnp.int32)

        bgl = (
            jax.lax.broadcasted_iota(jnp.int32, (_V_BLK, _B_BLK), 1)
            + s * _B_BLK
        )
        lin = bgl * v_total + vgl
        g = _gumbel_bits(lin.astype(jnp.uint32))
        sc = g + logits_t_ref[...]
        sc = jnp.where(vgl < v_total, sc, -jnp.inf)

        m = jnp.max(sc, axis=0, keepdims=True)
        cand = jnp.where(sc == m, vgl, jnp.int32(2**31 - 1))
        li = jnp.min(cand, axis=0, keepdims=True)

        upd = m > cur_v[...]
        cur_v[...] = jnp.where(upd, m, cur_v[...])
        cur_i[...] = jnp.where(upd, li, cur_i[...])

    @pl.when(s > 0)
    def _write():
        prev_i = best_idx.at[1 - par]
        out_t_ref[...] = (vgl == prev_i[...]).astype(jnp.float32)


def kernel(logits):
    b, v = logits.shape
    n_bb = pl.cdiv(b, _B_BLK)
    n_vb = pl.cdiv(v, _V_BLK)

    def in_map(s, j):
        return (jnp.where(s < n_bb, j, 0), jnp.minimum(s, n_bb - 1))

    def out_map(s, j):
        return (jnp.where(s > 0, j, 0), jnp.maximum(s - 1, 0))

    out_t = pl.pallas_call(
        lambda *refs: _body(n_bb, v, *refs),
        grid=(n_bb + 1, n_vb),
        in_specs=[pl.BlockSpec((_V_BLK, _B_BLK), in_map)],
        out_specs=pl.BlockSpec((_V_BLK, _B_BLK), out_map),
        out_shape=jax.ShapeDtypeStruct((v, b), jnp.float32),
        scratch_shapes=[
            pltpu.VMEM((2, 1, _B_BLK), jnp.float32),
            pltpu.VMEM((2, 1, _B_BLK), jnp.int32),
        ],
        compiler_params=pltpu.CompilerParams(
            dimension_semantics=("arbitrary", "arbitrary"),
        ),
    )(logits.T)
    return out_t.T


# speedup vs baseline: 1.2032x; 1.2032x over previous
import numpy as np

import jax
import jax.numpy as jnp
from jax.experimental import pallas as pl
from jax.experimental.pallas import tpu as pltpu

_B_BLK = 256
_V_BLK = 2048

_KS0 = np.uint32(0)
_KS1 = np.uint32(42)
_KS2 = np.uint32(np.uint32(0x1BD11BDA) ^ np.uint32(42))
_ROT = ((13, 15, 26, 6), (17, 29, 16, 24))
_TINY = np.float32(np.finfo(np.float32).tiny)


def _gumbel_bits(lin_u32):
    ks = (_KS0, _KS1, _KS2)
    x0 = jnp.zeros_like(lin_u32)
    x1 = lin_u32 + _KS1

    for r in range(5):
        for d in _ROT[r % 2]:
            x0 = x0 + x1
            x1 = (x1 << np.uint32(d)) | (x1 >> np.uint32(32 - d))
            x1 = x0 ^ x1
        x0 = x0 + ks[(r + 1) % 3]
        x1 = x1 + ks[(r + 2) % 3] + np.uint32(r + 1)

    bits = x0 ^ x1
    fb = (bits >> np.uint32(9)) | np.uint32(0x3F800000)
    u = jax.lax.bitcast_convert_type(fb, jnp.float32) - jnp.float32(1.0)
    u = jnp.maximum(_TINY, u * (np.float32(1.0) - _TINY) + _TINY)
    return -jnp.log(-jnp.log(u))


def _body(n_bb, v_total, logits_ref, out_ref, best_val, best_idx):
    s = pl.program_id(0)
    vb = pl.program_id(1)
    par = jax.lax.rem(s, 2)

    @pl.when(s < n_bb)
    def _sample():
        cur_v = best_val.at[par]
        cur_i = best_idx.at[par]

        @pl.when(vb == 0)
        def _init():
            cur_v[...] = jnp.full((_B_BLK, 1), -jnp.inf, jnp.float32)
            cur_i[...] = jnp.zeros((_B_BLK, 1), jnp.int32)

        rows = (
            jax.lax.broadcasted_iota(jnp.int32, (_B_BLK, _V_BLK), 0)
            + s * _B_BLK
        )
        cols = (
            jax.lax.broadcasted_iota(jnp.int32, (_B_BLK, _V_BLK), 1)
            + vb * _V_BLK
        )
        lin = rows * v_total + cols
        g = _gumbel_bits(lin.astype(jnp.uint32))
        sc = g + logits_ref[...]
        sc = jnp.where(cols < v_total, sc, -jnp.inf)

        m = jnp.max(sc, axis=1, keepdims=True)
        cand = jnp.where(sc == m, cols, jnp.int32(2**31 - 1))
        li = jnp.min(cand, axis=1, keepdims=True)

        upd = m > cur_v[...]
        cur_v[...] = jnp.where(upd, m, cur_v[...])
        cur_i[...] = jnp.where(upd, li, cur_i[...])

    @pl.when(s > 0)
    def _write():
        prev_i = best_idx.at[1 - par]
        cols = (
            jax.lax.broadcasted_iota(jnp.int32, (_B_BLK, _V_BLK), 1)
            + vb * _V_BLK
        )
        out_ref[...] = (cols == prev_i[...]).astype(jnp.float32)


def kernel(logits):
    b, v = logits.shape
    n_bb = pl.cdiv(b, _B_BLK)
    n_vb = pl.cdiv(v, _V_BLK)

    def in_map(s, j):
        return (jnp.minimum(s, n_bb - 1), jnp.where(s < n_bb, j, 0))

    def out_map(s, j):
        return (jnp.maximum(s - 1, 0), jnp.where(s > 0, j, 0))

    out = pl.pallas_call(
        lambda *refs: _body(n_bb, v, *refs),
        grid=(n_bb + 1, n_vb),
        in_specs=[pl.BlockSpec((_B_BLK, _V_BLK), in_map)],
        out_specs=pl.BlockSpec((_B_BLK, _V_BLK), out_map),
        out_shape=jax.ShapeDtypeStruct((b, v), jnp.float32),
        scratch_shapes=[
            pltpu.VMEM((2, _B_BLK, 1), jnp.float32),
            pltpu.VMEM((2, _B_BLK, 1), jnp.int32),
        ],
        compiler_params=pltpu.CompilerParams(
            dimension_semantics=("arbitrary", "arbitrary"),
        ),
    )(logits)
    return out
